# Initial kernel scaffold; baseline (speedup 1.0000x reference)
import jax
import jax.numpy as jnp
from jax.experimental import pallas as pl


def kernel(x, edge_index, edge_type, bases1, coefs1, bases2, coefs2, bases3, coefs3):
    raise NotImplementedError("write your pallas kernel here")


# speedup vs baseline: 24.1765x; 24.1765x over previous
import functools

import jax
import jax.numpy as jnp
from jax import lax
from jax.experimental import pallas as pl
from jax.experimental.pallas import tpu as pltpu
from jax.experimental.pallas import tpu_sc as plsc

NC = 2
NS = 16
NW = NC * NS
KD = 80


def _mesh():
    return plsc.VectorSubcoreMesh(
        core_axis_name="c", subcore_axis_name="s", num_cores=NC, num_subcores=NS
    )


def _sc_norm(gdst3, RNP, CD):
    def body(gdst_ref, out_ref, idx2, gidx, vals, tbl, zbuf, ones, acc):
        cid = lax.axis_index("c")
        sid = lax.axis_index("s")
        wid = cid * NS + sid
        ZB = RNP // NS
        z16 = jnp.zeros((16,), jnp.float32)

        def zfill(i, _):
            zbuf[pl.ds(i * 16, 16)] = z16
            return 0

        lax.fori_loop(0, ZB // 16, zfill, 0)
        for j in range(KD // 16):
            ones[pl.ds(j * 16, 16)] = jnp.ones((16,), jnp.float32)
        pltpu.sync_copy(zbuf, acc.at[pl.ds(sid * ZB, ZB)])
        plsc.subcore_barrier()
        pltpu.sync_copy(gdst_ref.at[2 * sid], idx2.at[0])
        pltpu.sync_copy(gdst_ref.at[2 * sid + 1], idx2.at[1])
        for b in range(2):
            def addchunk(c, _, b=b):
                pltpu.sync_copy(ones, acc.at[idx2.at[b, c]], add=True)
                return 0

            lax.fori_loop(0, CD, addchunk, 0)
        plsc.subcore_barrier()
        pltpu.sync_copy(acc, tbl)
        pltpu.sync_copy(gdst_ref.at[wid], gidx)

        def gchunk(c, _):
            def grow(j, _):
                iv = gidx[c, pl.ds(j * 16, 16)]
                dv = plsc.load_gather(tbl, [iv])
                vals[c, pl.ds(j * 16, 16)] = 1.0 / jnp.maximum(dv, 1.0)
                return 0

            lax.fori_loop(0, KD // 16, grow, 0)
            return 0

        lax.fori_loop(0, CD, gchunk, 0)
        pltpu.sync_copy(vals, out_ref.at[wid])

    return pl.kernel(
        body,
        out_type=jax.ShapeDtypeStruct((NW, CD, KD), jnp.float32),
        mesh=_mesh(),
        compiler_params=pltpu.CompilerParams(needs_layout_passes=False, use_tc_tiling_on_sc=False),
        scratch_types=[
            pltpu.VMEM((2, CD, KD), jnp.int32),
            pltpu.VMEM((CD, KD), jnp.int32),
            pltpu.VMEM((CD, KD), jnp.float32),
            pltpu.VMEM((RNP,), jnp.float32),
            pltpu.VMEM((RNP // NS,), jnp.float32),
            pltpu.VMEM((KD,), jnp.float32),
            pltpu.VMEM_SHARED((RNP,), jnp.float32),
        ],
    )(gdst3)


def _sc_agg(table, gsrc3, dst3, nrm3, N, D, CD):
    NPT = N // NS
    ZR = 32
    assert NPT % ZR == 0

    def body(table_ref, gsrc_ref, dst_ref, nrm_ref, out_ref,
             gidx, didx, nrm, rows, zrows, acc, gsem):
        cid = lax.axis_index("c")
        sid = lax.axis_index("s")
        wid = cid * NS + sid

        def zfill(i, _):
            for v in range(D // 16):
                zrows[i, pl.ds(v * 16, 16)] = jnp.zeros((16,), jnp.float32)
            return 0

        lax.fori_loop(0, ZR, zfill, 0)
        for kk in range(NPT // ZR):
            pltpu.sync_copy(zrows, acc.at[pl.ds(sid * NPT + kk * ZR, ZR)])
        plsc.subcore_barrier()
        pltpu.sync_copy(gsrc_ref.at[wid], gidx)
        pltpu.sync_copy(dst_ref.at[wid], didx)
        pltpu.sync_copy(nrm_ref.at[wid], nrm)

        def chunk(c, _):
            pltpu.async_copy(table_ref.at[gidx.at[c]], rows, gsem).wait()
            for k in range(KD):
                sv = plsc.load_gather(
                    nrm,
                    [jnp.full((16,), c, jnp.int32),
                     jnp.full((16,), k, jnp.int32)],
                )
                for v in range(D // 16):
                    rows[k, pl.ds(v * 16, 16)] = rows[k, pl.ds(v * 16, 16)] * sv
            pltpu.sync_copy(rows, acc.at[didx.at[c]], add=True)
            return 0

        lax.fori_loop(0, CD, chunk, 0)
        plsc.subcore_barrier()
        for kk in range(NPT // ZR):
            pltpu.sync_copy(
                acc.at[pl.ds(sid * NPT + kk * ZR, ZR)],
                out_ref.at[cid, pl.ds(sid * NPT + kk * ZR, ZR)],
            )

    return pl.kernel(
        body,
        out_type=jax.ShapeDtypeStruct((NC, N, D), jnp.float32),
        mesh=_mesh(),
        compiler_params=pltpu.CompilerParams(needs_layout_passes=False, use_tc_tiling_on_sc=False),
        scratch_types=[
            pltpu.VMEM((CD, KD), jnp.int32),
            pltpu.VMEM((CD, KD), jnp.int32),
            pltpu.VMEM((CD, KD), jnp.float32),
            pltpu.VMEM((KD, D), jnp.float32),
            pltpu.VMEM((ZR, D), jnp.float32),
            pltpu.VMEM_SHARED((N, D), jnp.float32),
            pltpu.SemaphoreType.DMA,
        ],
    )(table, gsrc3, dst3, nrm3)


def _tc_xw(x, bases, coefs, first):
    R, NB = coefs.shape
    Din, Dout = bases.shape[1], bases.shape[2]
    N = x.shape[0] if first else x.shape[1]
    BN = 512
    assert N % BN == 0

    def body(x_ref, b_ref, c_ref, o_ref):
        if first:
            h = x_ref[...]
        else:
            h = jnp.maximum(x_ref[0] + x_ref[1], 0.0)
        for r in range(R):
            W = sum(c_ref[r, b] * b_ref[b] for b in range(NB))
            o_ref[r] = lax.dot_general(
                h, W, (((1,), (0,)), ((), ())),
                precision=lax.Precision.HIGHEST,
                preferred_element_type=jnp.float32,
            )

    if first:
        x_spec = pl.BlockSpec((BN, Din), lambda i: (i, 0))
    else:
        x_spec = pl.BlockSpec((2, BN, Din), lambda i: (0, i, 0))
    return pl.pallas_call(
        body,
        grid=(N // BN,),
        in_specs=[
            x_spec,
            pl.BlockSpec((NB, Din, Dout), lambda i: (0, 0, 0)),
            pl.BlockSpec((R, NB), lambda i: (0, 0)),
        ],
        out_specs=pl.BlockSpec((R, BN, Dout), lambda i: (0, i, 0)),
        out_shape=jax.ShapeDtypeStruct((R, N, Dout), jnp.float32),
    )(x, bases, coefs)


def _tc_final(p, N):
    def body(p_ref, o_ref):
        o_ref[...] = jnp.maximum(p_ref[0, :N] + p_ref[1, :N], 0.0)

    return pl.pallas_call(
        body,
        out_shape=jax.ShapeDtypeStruct((N, p.shape[2]), jnp.float32),
    )(p)


def kernel(x, edge_index, edge_type, bases1, coefs1, bases2, coefs2,
           bases3, coefs3):
    N, H = x.shape
    E = edge_index.shape[1]
    R = coefs1.shape[0]
    OUT = bases3.shape[2]
    NP = ((N + 2047) // 2048) * 2048
    RNP = R * NP
    CD = E // (NW * KD)
    assert CD * NW * KD == E

    src = edge_index[0]
    dst = edge_index[1]
    et = edge_type
    gsrc3 = (et * NP + src).reshape(NW, CD, KD)
    gdst3 = (et * NP + dst).reshape(NW, CD, KD)
    dst3 = dst.reshape(NW, CD, KD)

    nrm3 = _sc_norm(gdst3, RNP, CD)

    xp = jnp.pad(x, ((0, NP - N), (0, 0)))
    t = _tc_xw(xp, bases1, coefs1, first=True).reshape(RNP, H)
    p = _sc_agg(t, gsrc3, dst3, nrm3, NP, H, CD)
    t = _tc_xw(p, bases2, coefs2, first=False).reshape(RNP, H)
    p = _sc_agg(t, gsrc3, dst3, nrm3, NP, H, CD)
    t = _tc_xw(p, bases3, coefs3, first=False).reshape(RNP, OUT)
    p = _sc_agg(t, gsrc3, dst3, nrm3, NP, OUT, CD)
    return _tc_final(p, N)


# speedup vs baseline: 26.1671x; 1.0823x over previous
import functools

import jax
import jax.numpy as jnp
from jax import lax
from jax.experimental import pallas as pl
from jax.experimental.pallas import tpu as pltpu
from jax.experimental.pallas import tpu_sc as plsc

NC = 2
NS = 16
NW = NC * NS
KD = 80


def _mesh():
    return plsc.VectorSubcoreMesh(
        core_axis_name="c", subcore_axis_name="s", num_cores=NC, num_subcores=NS
    )


def _sc_norm(gdst3, RNP, CD):
    def body(gdst_ref, out_ref, idx2, gidx, vals, tbl, zbuf, ones, acc):
        cid = lax.axis_index("c")
        sid = lax.axis_index("s")
        wid = cid * NS + sid
        ZB = RNP // NS
        z16 = jnp.zeros((16,), jnp.float32)

        def zfill(i, _):
            zbuf[pl.ds(i * 16, 16)] = z16
            return 0

        lax.fori_loop(0, ZB // 16, zfill, 0)
        for j in range(KD // 16):
            ones[pl.ds(j * 16, 16)] = jnp.ones((16,), jnp.float32)
        pltpu.sync_copy(zbuf, acc.at[pl.ds(sid * ZB, ZB)])
        plsc.subcore_barrier()
        pltpu.sync_copy(gdst_ref.at[2 * sid], idx2.at[0])
        pltpu.sync_copy(gdst_ref.at[2 * sid + 1], idx2.at[1])
        for b in range(2):
            def addchunk(c, _, b=b):
                pltpu.sync_copy(ones, acc.at[idx2.at[b, c]], add=True)
                return 0

            lax.fori_loop(0, CD, addchunk, 0)
        plsc.subcore_barrier()
        pltpu.sync_copy(acc, tbl)
        pltpu.sync_copy(gdst_ref.at[wid], gidx)

        def gchunk(c, _):
            def grow(j, _):
                iv = gidx[c, pl.ds(j * 16, 16)]
                dv = plsc.load_gather(tbl, [iv])
                vals[c, pl.ds(j * 16, 16)] = 1.0 / jnp.maximum(dv, 1.0)
                return 0

            lax.fori_loop(0, KD // 16, grow, 0)
            return 0

        lax.fori_loop(0, CD, gchunk, 0)
        pltpu.sync_copy(vals, out_ref.at[wid])

    return pl.kernel(
        body,
        out_type=jax.ShapeDtypeStruct((NW, CD, KD), jnp.float32),
        mesh=_mesh(),
        compiler_params=pltpu.CompilerParams(needs_layout_passes=False, use_tc_tiling_on_sc=False),
        scratch_types=[
            pltpu.VMEM((2, CD, KD), jnp.int32),
            pltpu.VMEM((CD, KD), jnp.int32),
            pltpu.VMEM((CD, KD), jnp.float32),
            pltpu.VMEM((RNP,), jnp.float32),
            pltpu.VMEM((RNP // NS,), jnp.float32),
            pltpu.VMEM((KD,), jnp.float32),
            pltpu.VMEM_SHARED((RNP,), jnp.float32),
        ],
    )(gdst3)


def _sc_agg(table, packed, N, D, CA, KA):
    NPT = N // NS
    ZR = 32
    RD, PD = 3, 5
    assert NPT % ZR == 0

    def body(table_ref, pk_ref, out_ref, pk, rows, zrows, acc,
             psem, gsem, ssem):
        cid = lax.axis_index("c")
        sid = lax.axis_index("s")
        wid = cid * NS + sid

        def zfill(i, _):
            for v in range(D // 16):
                zrows[i, pl.ds(v * 16, 16)] = jnp.zeros((16,), jnp.float32)
            return 0

        lax.fori_loop(0, ZR, zfill, 0)
        for kk in range(NPT // ZR):
            pltpu.sync_copy(zrows, acc.at[pl.ds(sid * NPT + kk * ZR, ZR)])
        plsc.subcore_barrier()

        def start_pk(c):
            pltpu.async_copy(pk_ref.at[wid, c], pk.at[lax.rem(c, PD)], psem)

        def wait_pk():
            pltpu.make_async_copy(pk_ref.at[0, 0], pk.at[0], psem).wait()

        def start_gather(c):
            pltpu.async_copy(
                table_ref.at[pk.at[lax.rem(c, PD), 0]],
                rows.at[lax.rem(c, RD)], gsem)

        def wait_gather():
            pltpu.make_async_copy(
                table_ref.at[pl.ds(0, KA)], rows.at[0], gsem).wait()

        def start_scatter(c):
            pltpu.async_copy(
                rows.at[lax.rem(c, RD)],
                acc.at[pk.at[lax.rem(c, PD), 1]], ssem, add=True)

        def wait_scatter():
            pltpu.make_async_copy(
                table_ref.at[pl.ds(0, KA)], rows.at[0], ssem).wait()

        for c in range(min(4, CA)):
            start_pk(c)
        wait_pk()
        start_gather(0)
        if CA > 1:
            wait_pk()
            start_gather(1)

        def step(c, _):
            @pl.when(c >= 1)
            def _():
                wait_scatter()

            @pl.when(c + 2 < CA)
            def _():
                wait_pk()
                start_gather(c + 2)

            @pl.when(c + 4 < CA)
            def _():
                start_pk(c + 4)

            wait_gather()
            ps = lax.rem(c, PD)
            rs = lax.rem(c, RD)
            for k in range(KA):
                nv = plsc.load_gather(
                    pk,
                    [jnp.full((16,), ps, jnp.int32),
                     jnp.full((16,), 2, jnp.int32),
                     jnp.full((16,), k, jnp.int32)])
                sv = plsc.bitcast(nv, jnp.float32)
                for v in range(D // 16):
                    rows[rs, k, pl.ds(v * 16, 16)] = (
                        rows[rs, k, pl.ds(v * 16, 16)] * sv)
            start_scatter(c)
            return 0

        lax.fori_loop(0, CA, step, 0)
        wait_scatter()
        plsc.subcore_barrier()
        for kk in range(NPT // ZR):
            pltpu.sync_copy(
                acc.at[pl.ds(sid * NPT + kk * ZR, ZR)],
                out_ref.at[cid, pl.ds(sid * NPT + kk * ZR, ZR)],
            )

    return pl.kernel(
        body,
        out_type=jax.ShapeDtypeStruct((NC, N, D), jnp.float32),
        mesh=_mesh(),
        compiler_params=pltpu.CompilerParams(
            needs_layout_passes=False, use_tc_tiling_on_sc=False),
        scratch_types=[
            pltpu.VMEM((PD, 3, KA), jnp.int32),
            pltpu.VMEM((RD, KA, D), jnp.float32),
            pltpu.VMEM((ZR, D), jnp.float32),
            pltpu.VMEM_SHARED((N, D), jnp.float32),
            pltpu.SemaphoreType.DMA,
            pltpu.SemaphoreType.DMA,
            pltpu.SemaphoreType.DMA,
        ],
    )(table, packed)


def _tc_xw(x, bases, coefs, first):
    R, NB = coefs.shape
    Din, Dout = bases.shape[1], bases.shape[2]
    N = x.shape[0] if first else x.shape[1]
    BN = 512
    assert N % BN == 0

    def body(x_ref, b_ref, c_ref, o_ref):
        if first:
            h = x_ref[...]
        else:
            h = jnp.maximum(x_ref[0] + x_ref[1], 0.0)
        for r in range(R):
            W = sum(c_ref[r, b] * b_ref[b] for b in range(NB))
            o_ref[r] = lax.dot_general(
                h, W, (((1,), (0,)), ((), ())),
                precision=lax.Precision.HIGHEST,
                preferred_element_type=jnp.float32,
            )

    if first:
        x_spec = pl.BlockSpec((BN, Din), lambda i: (i, 0))
    else:
        x_spec = pl.BlockSpec((2, BN, Din), lambda i: (0, i, 0))
    return pl.pallas_call(
        body,
        grid=(N // BN,),
        in_specs=[
            x_spec,
            pl.BlockSpec((NB, Din, Dout), lambda i: (0, 0, 0)),
            pl.BlockSpec((R, NB), lambda i: (0, 0)),
        ],
        out_specs=pl.BlockSpec((R, BN, Dout), lambda i: (0, i, 0)),
        out_shape=jax.ShapeDtypeStruct((R, N, Dout), jnp.float32),
    )(x, bases, coefs)


def _tc_final(p, N):
    def body(p_ref, o_ref):
        o_ref[...] = jnp.maximum(p_ref[0, :N] + p_ref[1, :N], 0.0)

    return pl.pallas_call(
        body,
        out_shape=jax.ShapeDtypeStruct((N, p.shape[2]), jnp.float32),
    )(p)


def kernel(x, edge_index, edge_type, bases1, coefs1, bases2, coefs2,
           bases3, coefs3):
    N, H = x.shape
    E = edge_index.shape[1]
    R = coefs1.shape[0]
    OUT = bases3.shape[2]
    NP = ((N + 2047) // 2048) * 2048
    RNP = R * NP

    src = edge_index[0]
    dst = edge_index[1]
    et = edge_type
    gsrc = et * NP + src
    gdst = et * NP + dst
    KN, KA = 80, 100
    CN = E // (NW * KN)
    CA = E // (NW * KA)

    nrm3 = _sc_norm(gdst.reshape(NW, CN, KN), RNP, CN)
    packed = jnp.stack(
        [gsrc.reshape(NW, CA, KA),
         dst.reshape(NW, CA, KA),
         lax.bitcast_convert_type(nrm3.reshape(NW, CA, KA), jnp.int32)],
        axis=2,
    )

    xp = jnp.pad(x, ((0, NP - N), (0, 0)))
    t = _tc_xw(xp, bases1, coefs1, first=True).reshape(RNP, H)
    p = _sc_agg(t, packed, NP, H, CA, KA)
    t = _tc_xw(p, bases2, coefs2, first=False).reshape(RNP, H)
    p = _sc_agg(t, packed, NP, H, CA, KA)
    t = _tc_xw(p, bases3, coefs3, first=False).reshape(RNP, OUT)
    p = _sc_agg(t, packed, NP, OUT, CA, KA)
    return _tc_final(p, N)
